# Initial kernel scaffold; baseline (speedup 1.0000x reference)
import jax
import jax.numpy as jnp
from jax.experimental import pallas as pl


def kernel(x, gate_W, gate_b, W1, b1, W2, b2):
    raise NotImplementedError("write your pallas kernel here")


# speedup vs baseline: 2.5674x; 2.5674x over previous
import jax
import jax.numpy as jnp
from jax.experimental import pallas as pl
from jax.experimental.pallas import tpu as pltpu

DHC = 512


def _moe_dense_kernel(x_ref, gw_ref, gb_ref, w1_ref, b1_ref, w2_ref, b2_ref,
                      out_ref, p_scr):
    e = pl.program_id(0)
    j = pl.program_id(1)
    S, E = p_scr.shape

    @pl.when((e == 0) & (j == 0))
    def _gate():
        x = x_ref[...]
        logits = jax.lax.dot_general(
            x, gw_ref[...], (((1,), (1,)), ((), ())),
            preferred_element_type=jnp.float32) + gb_ref[...]
        lane = jax.lax.broadcasted_iota(jnp.int32, (S, E), 1)
        m0 = jnp.max(logits, axis=1, keepdims=True)
        i0 = jnp.min(jnp.where(logits == m0, lane, E), axis=1, keepdims=True)
        l1 = jnp.where(lane == i0, -jnp.inf, logits)
        m1 = jnp.max(l1, axis=1, keepdims=True)
        i1 = jnp.min(jnp.where(l1 == m1, lane, E), axis=1, keepdims=True)
        e1 = jnp.exp(m1 - m0)
        w0 = 1.0 / (1.0 + e1)
        w1 = 1.0 - w0
        p_scr[...] = jnp.where(lane == i0, w0, 0.0) + jnp.where(lane == i1, w1, 0.0)

    x = x_ref[...]
    h = x @ w1_ref[0]
    h = h + b1_ref[0]
    h = 0.5 * h * (1.0 + jax.lax.erf(h * 0.7071067811865476))
    y = jax.lax.dot_general(h, w2_ref[0], (((1,), (0,)), ((), ())),
                            preferred_element_type=jnp.float32)

    y = jnp.where(j == 0, y + b2_ref[0], y)

    lane = jax.lax.broadcasted_iota(jnp.int32, p_scr.shape, 1)
    wi = jnp.sum(jnp.where(lane == e, p_scr[...], 0.0), axis=1, keepdims=True)
    contrib = y * wi

    @pl.when((e == 0) & (j == 0))
    def _init():
        out_ref[...] = contrib

    @pl.when(~((e == 0) & (j == 0)))
    def _acc():
        out_ref[...] += contrib


def kernel(x, gate_W, gate_b, W1, b1, W2, b2):
    B, S, D = x.shape
    E, _, DH = W1.shape
    x_flat = x.reshape(S, D)
    gb = gate_b.reshape(1, E)
    b1r = b1.reshape(E, 1, DH)
    b2r = b2.reshape(E, 1, D)
    nj = DH // DHC

    out = pl.pallas_call(
        _moe_dense_kernel,
        grid=(E, nj),
        in_specs=[
            pl.BlockSpec((S, D), lambda e, j: (0, 0)),
            pl.BlockSpec((E, D), lambda e, j: (0, 0)),
            pl.BlockSpec((1, E), lambda e, j: (0, 0)),
            pl.BlockSpec((1, D, DHC), lambda e, j: (e, 0, j)),
            pl.BlockSpec((1, 1, DHC), lambda e, j: (e, 0, j)),
            pl.BlockSpec((1, DHC, D), lambda e, j: (e, j, 0)),
            pl.BlockSpec((1, 1, D), lambda e, j: (e, 0, 0)),
        ],
        out_specs=pl.BlockSpec((S, D), lambda e, j: (0, 0)),
        out_shape=jax.ShapeDtypeStruct((S, D), jnp.float32),
        scratch_shapes=[pltpu.VMEM((S, E), jnp.float32)],
    )(x_flat, gate_W, gb, W1, b1r, W2, b2r)
    return out.reshape(B, S, D)
